# baseline (speedup 1.0000x reference)
import functools

import jax
import jax.numpy as jnp
from jax import lax
from jax.experimental import pallas as pl
from jax.experimental.pallas import tpu as pltpu
from jax.experimental.pallas import tpu_sc as plsc

B = 16384
D = 64
NUM_CORES = 2
NUM_SUBCORES = 16
NW = NUM_CORES * NUM_SUBCORES
BPW = B // NW
HALF = BPW // 2
MARGIN = 0.1


def _cf_body(uid_hbm, mid_hbm, ue_hbm, me_hbm, bu_hbm, bm_hbm, out_hbm,
             uidx_v, midx_v, pu_v, pm_v, u_pairs, m_pairs, bu_v, bm_v,
             out_v, sem, bsem):
    wid = lax.axis_index("s") * NUM_CORES + lax.axis_index("c")
    base = wid * BPW

    pltpu.sync_copy(uid_hbm.at[pl.ds(base, BPW)], uidx_v)
    pltpu.sync_copy(mid_hbm.at[pl.ds(base, BPW)], midx_v)

    cbu = pltpu.async_copy(bu_hbm.at[uidx_v], bu_v, bsem)
    cbm = pltpu.async_copy(bm_hbm.at[midx_v], bm_v, bsem)

    lane = lax.iota(jnp.int32, 16)

    for half in range(2):
        off = half * HALF

        def idx_body(i, carry):
            o = i * 16
            pu_v[pl.ds(o, 16)] = uidx_v[pl.ds(off + o, 16)] >> 1
            pm_v[pl.ds(o, 16)] = midx_v[pl.ds(off + o, 16)] >> 1
            return carry

        lax.fori_loop(0, HALF // 16, idx_body, 0)

        cu = pltpu.async_copy(ue_hbm.at[pu_v], u_pairs, sem)
        cm = pltpu.async_copy(me_hbm.at[pm_v], m_pairs, sem)
        cu.wait()
        cm.wait()
        if half == 0:
            cbu.wait()
            cbm.wait()

        def chunk_body(c, carry):
            o = c * 16
            hu16 = (uidx_v[pl.ds(off + o, 16)] & 1) * D
            hm16 = (midx_v[pl.ds(off + o, 16)] & 1) * D
            acc = jnp.zeros((16,), jnp.float32)
            for k in range(16):
                r = o + k
                hu = hu16[k]
                hm = hm16[k]
                p = (u_pairs[r, pl.ds(hu, 16)] * m_pairs[r, pl.ds(hm, 16)]
                     + u_pairs[r, pl.ds(hu + 16, 16)]
                     * m_pairs[r, pl.ds(hm + 16, 16)]
                     + u_pairs[r, pl.ds(hu + 32, 16)]
                     * m_pairs[r, pl.ds(hm + 32, 16)]
                     + u_pairs[r, pl.ds(hu + 48, 16)]
                     * m_pairs[r, pl.ds(hm + 48, 16)])
                acc = jnp.where(lane == k, jnp.sum(p), acc)
            x = acc + bu_v[pl.ds(off + o, 16)] + bm_v[pl.ds(off + o, 16)]
            y = 1.0 / (1.0 + jnp.exp(-x))
            out_v[pl.ds(off + o, 16)] = y * (1.0 + 2.0 * MARGIN) - MARGIN
            return carry

        lax.fori_loop(0, HALF // 16, chunk_body, 0)

    pltpu.sync_copy(out_v, out_hbm.at[pl.ds(base, BPW)])


@functools.partial(
    pl.kernel,
    out_type=jax.ShapeDtypeStruct((B,), jnp.float32),
    mesh=plsc.VectorSubcoreMesh(core_axis_name="c", subcore_axis_name="s"),
    compiler_params=pltpu.CompilerParams(needs_layout_passes=False),
    scratch_types=[
        pltpu.VMEM((BPW,), jnp.int32),
        pltpu.VMEM((BPW,), jnp.int32),
        pltpu.VMEM((HALF,), jnp.int32),
        pltpu.VMEM((HALF,), jnp.int32),
        pltpu.VMEM((HALF, 2 * D), jnp.float32),
        pltpu.VMEM((HALF, 2 * D), jnp.float32),
        pltpu.VMEM((BPW,), jnp.float32),
        pltpu.VMEM((BPW,), jnp.float32),
        pltpu.VMEM((BPW,), jnp.float32),
        pltpu.SemaphoreType.DMA,
        pltpu.SemaphoreType.DMA,
    ],
)
def _cf_kernel(*args):
    _cf_body(*args)


def kernel(user_id, movie_id, emb_users, emb_movies, bias_user, bias_movie):
    return _cf_kernel(
        user_id.astype(jnp.int32),
        movie_id.astype(jnp.int32),
        emb_users.reshape(-1, 2 * D),
        emb_movies.reshape(-1, 2 * D),
        bias_user.reshape(-1),
        bias_movie.reshape(-1),
    )


# speedup vs baseline: 1.0092x; 1.0092x over previous
import functools

import jax
import jax.numpy as jnp
from jax import lax
from jax.experimental import pallas as pl
from jax.experimental.pallas import tpu as pltpu
from jax.experimental.pallas import tpu_sc as plsc

B = 16384
D = 64
NUM_CORES = 2
NUM_SUBCORES = 16
NW = NUM_CORES * NUM_SUBCORES
BPW = B // NW
MARGIN = 0.1


def _cf_body(uid_hbm, mid_hbm, ue_hbm, me_hbm, bu_hbm, bm_hbm, out_hbm,
             uidx_v, midx_v, u_rows, m_rows, bu_v, bm_v, out_v, sem):
    wid = lax.axis_index("s") * NUM_CORES + lax.axis_index("c")
    base = wid * BPW

    pltpu.sync_copy(uid_hbm.at[pl.ds(base, BPW)], uidx_v)
    pltpu.sync_copy(mid_hbm.at[pl.ds(base, BPW)], midx_v)

    cu = pltpu.async_copy(ue_hbm.at[uidx_v], u_rows, sem)
    cm = pltpu.async_copy(me_hbm.at[midx_v], m_rows, sem)
    cbu = pltpu.async_copy(bu_hbm.at[uidx_v], bu_v, sem)
    cbm = pltpu.async_copy(bm_hbm.at[midx_v], bm_v, sem)
    cu.wait()
    cm.wait()
    cbu.wait()
    cbm.wait()

    lane = lax.iota(jnp.int32, 16)

    def chunk_body(c, carry):
        o = c * 16
        acc = jnp.zeros((16,), jnp.float32)
        for k in range(16):
            r = o + k
            p = (u_rows[r, pl.ds(0, 16)] * m_rows[r, pl.ds(0, 16)]
                 + u_rows[r, pl.ds(16, 16)] * m_rows[r, pl.ds(16, 16)]
                 + u_rows[r, pl.ds(32, 16)] * m_rows[r, pl.ds(32, 16)]
                 + u_rows[r, pl.ds(48, 16)] * m_rows[r, pl.ds(48, 16)])
            acc = jnp.where(lane == k, jnp.sum(p), acc)
        x = acc + bu_v[pl.ds(o, 16)] + bm_v[pl.ds(o, 16)]
        y = 1.0 / (1.0 + jnp.exp(-x))
        out_v[pl.ds(o, 16)] = y * (1.0 + 2.0 * MARGIN) - MARGIN
        return carry

    lax.fori_loop(0, BPW // 16, chunk_body, 0)

    pltpu.sync_copy(out_v, out_hbm.at[pl.ds(base, BPW)])


@functools.partial(
    pl.kernel,
    out_type=jax.ShapeDtypeStruct((B,), jnp.float32),
    mesh=plsc.VectorSubcoreMesh(core_axis_name="c", subcore_axis_name="s"),
    compiler_params=pltpu.CompilerParams(
        needs_layout_passes=False,
        use_tc_tiling_on_sc=False,
        skip_device_barrier=True,
    ),
    scratch_types=[
        pltpu.VMEM((BPW,), jnp.int32),
        pltpu.VMEM((BPW,), jnp.int32),
        pltpu.VMEM((BPW, D), jnp.float32),
        pltpu.VMEM((BPW, D), jnp.float32),
        pltpu.VMEM((BPW,), jnp.float32),
        pltpu.VMEM((BPW,), jnp.float32),
        pltpu.VMEM((BPW,), jnp.float32),
        pltpu.SemaphoreType.DMA,
    ],
)
def _cf_kernel(*args):
    _cf_body(*args)


def kernel(user_id, movie_id, emb_users, emb_movies, bias_user, bias_movie):
    return _cf_kernel(
        user_id.astype(jnp.int32),
        movie_id.astype(jnp.int32),
        emb_users,
        emb_movies,
        bias_user.reshape(-1),
        bias_movie.reshape(-1),
    )


# speedup vs baseline: 1.0100x; 1.0008x over previous
import functools

import jax
import jax.numpy as jnp
from jax import lax
from jax.experimental import pallas as pl
from jax.experimental.pallas import tpu as pltpu
from jax.experimental.pallas import tpu_sc as plsc

B = 16384
D = 64
NUM_CORES = 2
NUM_SUBCORES = 16
NW = NUM_CORES * NUM_SUBCORES
BPW = B // NW
MARGIN = 0.1


def _cf_body(uid_hbm, mid_hbm, ue_hbm, me_hbm, bu_hbm, bm_hbm, out_hbm,
             uidx_v, midx_v, u_rows, m_rows, bu_v, bm_v, out_v, sem):
    wid = lax.axis_index("s") * NUM_CORES + lax.axis_index("c")
    base = wid * BPW

    pltpu.sync_copy(uid_hbm.at[pl.ds(base, BPW)], uidx_v)
    pltpu.sync_copy(mid_hbm.at[pl.ds(base, BPW)], midx_v)

    cu = pltpu.async_copy(ue_hbm.at[uidx_v], u_rows, sem)
    cm = pltpu.async_copy(me_hbm.at[midx_v], m_rows, sem)
    cbu = pltpu.async_copy(bu_hbm.at[uidx_v], bu_v, sem)
    cbm = pltpu.async_copy(bm_hbm.at[midx_v], bm_v, sem)
    cu.wait()
    cm.wait()
    cbu.wait()
    cbm.wait()

    lane = lax.iota(jnp.int32, 16)

    def chunk_body(c, carry):
        o = c * 16

        def row_body(k, acc):
            r = o + k
            p = (u_rows[r, pl.ds(0, 16)] * m_rows[r, pl.ds(0, 16)]
                 + u_rows[r, pl.ds(16, 16)] * m_rows[r, pl.ds(16, 16)]
                 + u_rows[r, pl.ds(32, 16)] * m_rows[r, pl.ds(32, 16)]
                 + u_rows[r, pl.ds(48, 16)] * m_rows[r, pl.ds(48, 16)])
            return jnp.where(lane == k, jnp.sum(p), acc)

        acc = lax.fori_loop(0, 16, row_body, jnp.zeros((16,), jnp.float32))
        x = acc + bu_v[pl.ds(o, 16)] + bm_v[pl.ds(o, 16)]
        y = 1.0 / (1.0 + jnp.exp(-x))
        out_v[pl.ds(o, 16)] = y * (1.0 + 2.0 * MARGIN) - MARGIN
        return carry

    lax.fori_loop(0, BPW // 16, chunk_body, 0)

    pltpu.sync_copy(out_v, out_hbm.at[pl.ds(base, BPW)])


@functools.partial(
    pl.kernel,
    out_type=jax.ShapeDtypeStruct((B,), jnp.float32),
    mesh=plsc.VectorSubcoreMesh(core_axis_name="c", subcore_axis_name="s"),
    compiler_params=pltpu.CompilerParams(
        needs_layout_passes=False,
        use_tc_tiling_on_sc=False,
        skip_device_barrier=True,
    ),
    scratch_types=[
        pltpu.VMEM((BPW,), jnp.int32),
        pltpu.VMEM((BPW,), jnp.int32),
        pltpu.VMEM((BPW, D), jnp.float32),
        pltpu.VMEM((BPW, D), jnp.float32),
        pltpu.VMEM((BPW,), jnp.float32),
        pltpu.VMEM((BPW,), jnp.float32),
        pltpu.VMEM((BPW,), jnp.float32),
        pltpu.SemaphoreType.DMA,
    ],
)
def _cf_kernel(*args):
    _cf_body(*args)


def kernel(user_id, movie_id, emb_users, emb_movies, bias_user, bias_movie):
    return _cf_kernel(
        user_id.astype(jnp.int32),
        movie_id.astype(jnp.int32),
        emb_users,
        emb_movies,
        bias_user.reshape(-1),
        bias_movie.reshape(-1),
    )


# speedup vs baseline: 2.0366x; 2.0163x over previous
import functools

import jax
import jax.numpy as jnp
from jax import lax
from jax.experimental import pallas as pl
from jax.experimental.pallas import tpu as pltpu
from jax.experimental.pallas import tpu_sc as plsc

B = 16384
D = 64
NUM_CORES = 2
NUM_SUBCORES = 16
NW = NUM_CORES * NUM_SUBCORES
BPW = B // NW
MARGIN = 0.1


def _cf_body(uid_hbm, mid_hbm, ue_hbm, me_hbm, bu_hbm, bm_hbm, out_hbm,
             uidx_v, midx_v, u_st, m_st, bu_v, bm_v, out_v, sem, bsem):
    wid = lax.axis_index("s") * NUM_CORES + lax.axis_index("c")
    base = wid * BPW

    pltpu.sync_copy(uid_hbm.at[pl.ds(base, BPW)], uidx_v)
    pltpu.sync_copy(mid_hbm.at[pl.ds(base, BPW)], midx_v)

    cbu = pltpu.async_copy(bu_hbm.at[uidx_v], bu_v, bsem)
    cbm = pltpu.async_copy(bm_hbm.at[midx_v], bm_v, bsem)

    lane = lax.iota(jnp.int32, 16)

    def wave_body(w, carry):
        o = w * 16
        uid16 = uidx_v[pl.ds(o, 16)]
        mid16 = midx_v[pl.ds(o, 16)]
        gu16 = uid16 >> 3
        gm16 = mid16 >> 3
        ru16 = uid16 & 7
        rm16 = midx_v[pl.ds(o, 16)] & 7
        copies = []
        for k in range(16):
            copies.append(pltpu.async_copy(
                ue_hbm.at[gu16[k]], u_st.at[k], sem))
            copies.append(pltpu.async_copy(
                me_hbm.at[gm16[k]], m_st.at[k], sem))
        for c in copies:
            c.wait()

        acc = jnp.zeros((16,), jnp.float32)
        for k in range(16):
            ru = ru16[k]
            rm = rm16[k]
            p = (u_st[k, ru, pl.ds(0, 16)] * m_st[k, rm, pl.ds(0, 16)]
                 + u_st[k, ru, pl.ds(16, 16)] * m_st[k, rm, pl.ds(16, 16)]
                 + u_st[k, ru, pl.ds(32, 16)] * m_st[k, rm, pl.ds(32, 16)]
                 + u_st[k, ru, pl.ds(48, 16)] * m_st[k, rm, pl.ds(48, 16)])
            acc = jnp.where(lane == k, jnp.sum(p), acc)
        x = acc + bu_v[pl.ds(o, 16)] + bm_v[pl.ds(o, 16)]
        y = 1.0 / (1.0 + jnp.exp(-x))
        out_v[pl.ds(o, 16)] = y * (1.0 + 2.0 * MARGIN) - MARGIN
        return carry

    lax.fori_loop(0, BPW // 16, wave_body, 0)

    cbu.wait()
    cbm.wait()
    pltpu.sync_copy(out_v, out_hbm.at[pl.ds(base, BPW)])


@functools.partial(
    pl.kernel,
    out_type=jax.ShapeDtypeStruct((B,), jnp.float32),
    mesh=plsc.VectorSubcoreMesh(core_axis_name="c", subcore_axis_name="s"),
    compiler_params=pltpu.CompilerParams(
        needs_layout_passes=False,
        use_tc_tiling_on_sc=True,
    ),
    scratch_types=[
        pltpu.VMEM((BPW,), jnp.int32),
        pltpu.VMEM((BPW,), jnp.int32),
        pltpu.VMEM((16, 8, D), jnp.float32),
        pltpu.VMEM((16, 8, D), jnp.float32),
        pltpu.VMEM((BPW,), jnp.float32),
        pltpu.VMEM((BPW,), jnp.float32),
        pltpu.VMEM((BPW,), jnp.float32),
        pltpu.SemaphoreType.DMA,
        pltpu.SemaphoreType.DMA,
    ],
)
def _cf_kernel(*args):
    _cf_body(*args)


def kernel(user_id, movie_id, emb_users, emb_movies, bias_user, bias_movie):
    return _cf_kernel(
        user_id.astype(jnp.int32),
        movie_id.astype(jnp.int32),
        emb_users.reshape(-1, 8, D),
        emb_movies.reshape(-1, 8, D),
        bias_user.reshape(-1),
        bias_movie.reshape(-1),
    )


# speedup vs baseline: 2.1503x; 1.0558x over previous
import functools

import jax
import jax.numpy as jnp
from jax import lax
from jax.experimental import pallas as pl
from jax.experimental.pallas import tpu as pltpu
from jax.experimental.pallas import tpu_sc as plsc

B = 16384
D = 64
NUM_CORES = 2
NUM_SUBCORES = 16
NW = NUM_CORES * NUM_SUBCORES
BPW = B // NW
MARGIN = 0.1


def _cf_body(uid_hbm, mid_hbm, ue_hbm, me_hbm, bu_hbm, bm_hbm, out_hbm,
             uidx_v, midx_v, u_st, m_st, bu_v, bm_v, out_v,
             sem0, sem1, bsem):
    wid = lax.axis_index("s") * NUM_CORES + lax.axis_index("c")
    base = wid * BPW

    pltpu.sync_copy(uid_hbm.at[pl.ds(base, BPW)], uidx_v)
    pltpu.sync_copy(mid_hbm.at[pl.ds(base, BPW)], midx_v)

    cbu = pltpu.async_copy(bu_hbm.at[uidx_v], bu_v, bsem)
    cbm = pltpu.async_copy(bm_hbm.at[midx_v], bm_v, bsem)
    cbu.wait()
    cbm.wait()

    lane = lax.iota(jnp.int32, 16)
    NWAVES = BPW // 16

    def fire_wave(w, buf):
        o = w * 16
        gu16 = uidx_v[pl.ds(o, 16)] >> 3
        gm16 = midx_v[pl.ds(o, 16)] >> 3
        for k in range(16):
            pltpu.async_copy(ue_hbm.at[gu16[k]], u_st.at[buf, k], sem0)
            pltpu.async_copy(me_hbm.at[gm16[k]], m_st.at[buf, k], sem1)

    fire_wave(0, 0)

    def wave_body(w, carry):
        buf = w & 1

        @pl.when(w < NWAVES - 1)
        def _fire_next():
            fire_wave(w + 1, 1 - buf)

        for k in range(16):
            pltpu.make_async_copy(ue_hbm.at[0], u_st.at[buf, k], sem0).wait()
            pltpu.make_async_copy(me_hbm.at[0], m_st.at[buf, k], sem1).wait()

        o = w * 16
        ru16 = uidx_v[pl.ds(o, 16)] & 7
        rm16 = midx_v[pl.ds(o, 16)] & 7
        acc = jnp.zeros((16,), jnp.float32)
        for k in range(16):
            ru = ru16[k]
            rm = rm16[k]
            p = (u_st[buf, k, ru, pl.ds(0, 16)]
                 * m_st[buf, k, rm, pl.ds(0, 16)]
                 + u_st[buf, k, ru, pl.ds(16, 16)]
                 * m_st[buf, k, rm, pl.ds(16, 16)]
                 + u_st[buf, k, ru, pl.ds(32, 16)]
                 * m_st[buf, k, rm, pl.ds(32, 16)]
                 + u_st[buf, k, ru, pl.ds(48, 16)]
                 * m_st[buf, k, rm, pl.ds(48, 16)])
            acc = jnp.where(lane == k, jnp.sum(p), acc)
        x = acc + bu_v[pl.ds(o, 16)] + bm_v[pl.ds(o, 16)]
        y = 1.0 / (1.0 + jnp.exp(-x))
        out_v[pl.ds(o, 16)] = y * (1.0 + 2.0 * MARGIN) - MARGIN
        return carry

    lax.fori_loop(0, NWAVES, wave_body, 0)
    pltpu.sync_copy(out_v, out_hbm.at[pl.ds(base, BPW)])


@functools.partial(
    pl.kernel,
    out_type=jax.ShapeDtypeStruct((B,), jnp.float32),
    mesh=plsc.VectorSubcoreMesh(core_axis_name="c", subcore_axis_name="s"),
    compiler_params=pltpu.CompilerParams(
        needs_layout_passes=False,
        use_tc_tiling_on_sc=True,
    ),
    scratch_types=[
        pltpu.VMEM((BPW,), jnp.int32),
        pltpu.VMEM((BPW,), jnp.int32),
        pltpu.VMEM((2, 16, 8, D), jnp.float32),
        pltpu.VMEM((2, 16, 8, D), jnp.float32),
        pltpu.VMEM((BPW,), jnp.float32),
        pltpu.VMEM((BPW,), jnp.float32),
        pltpu.VMEM((BPW,), jnp.float32),
        pltpu.SemaphoreType.DMA,
        pltpu.SemaphoreType.DMA,
        pltpu.SemaphoreType.DMA,
    ],
)
def _cf_kernel(*args):
    _cf_body(*args)


def kernel(user_id, movie_id, emb_users, emb_movies, bias_user, bias_movie):
    return _cf_kernel(
        user_id.astype(jnp.int32),
        movie_id.astype(jnp.int32),
        emb_users.reshape(-1, 8, D),
        emb_movies.reshape(-1, 8, D),
        bias_user.reshape(-1),
        bias_movie.reshape(-1),
    )


# speedup vs baseline: 2.1510x; 1.0003x over previous
import functools

import jax
import jax.numpy as jnp
from jax import lax
from jax.experimental import pallas as pl
from jax.experimental.pallas import tpu as pltpu
from jax.experimental.pallas import tpu_sc as plsc

B = 16384
D = 64
NUM_CORES = 2
NUM_SUBCORES = 16
NW = NUM_CORES * NUM_SUBCORES
BPW = B // NW
MARGIN = 0.1


def _cf_body(uid_hbm, mid_hbm, ue_hbm, me_hbm, bu_hbm, bm_hbm, out_hbm,
             uidx_v, midx_v, u_st, m_st, bu_v, bm_v, out_v,
             sem0, sem1, bsem):
    wid = lax.axis_index("s") * NUM_CORES + lax.axis_index("c")
    base = wid * BPW

    pltpu.sync_copy(uid_hbm.at[pl.ds(base, BPW)], uidx_v)
    pltpu.sync_copy(mid_hbm.at[pl.ds(base, BPW)], midx_v)

    cbu = pltpu.async_copy(bu_hbm.at[uidx_v], bu_v, bsem)
    cbm = pltpu.async_copy(bm_hbm.at[midx_v], bm_v, bsem)
    cbu.wait()
    cbm.wait()

    lane = lax.iota(jnp.int32, 16)
    NWAVES = BPW // 16

    def fire_wave(w, buf):
        o = w * 16
        gu16 = uidx_v[pl.ds(o, 16)] >> 3
        gm16 = midx_v[pl.ds(o, 16)] >> 3
        for k in range(16):
            pltpu.async_copy(ue_hbm.at[gu16[k]], u_st.at[buf, k], sem0)
            pltpu.async_copy(me_hbm.at[gm16[k]], m_st.at[buf, k], sem1)

    fire_wave(0, 0)

    def wave_body(w, carry):
        buf = w & 1

        @pl.when(w < NWAVES - 1)
        def _fire_next():
            fire_wave(w + 1, 1 - buf)

        pltpu.make_async_copy(
            ue_hbm.at[pl.ds(0, 16)], u_st.at[buf], sem0).wait()
        pltpu.make_async_copy(
            me_hbm.at[pl.ds(0, 16)], m_st.at[buf], sem1).wait()

        o = w * 16
        ru16 = uidx_v[pl.ds(o, 16)] & 7
        rm16 = midx_v[pl.ds(o, 16)] & 7
        acc = jnp.zeros((16,), jnp.float32)
        for k in range(16):
            ru = ru16[k]
            rm = rm16[k]
            p = (u_st[buf, k, ru, pl.ds(0, 16)]
                 * m_st[buf, k, rm, pl.ds(0, 16)]
                 + u_st[buf, k, ru, pl.ds(16, 16)]
                 * m_st[buf, k, rm, pl.ds(16, 16)]
                 + u_st[buf, k, ru, pl.ds(32, 16)]
                 * m_st[buf, k, rm, pl.ds(32, 16)]
                 + u_st[buf, k, ru, pl.ds(48, 16)]
                 * m_st[buf, k, rm, pl.ds(48, 16)])
            acc = jnp.where(lane == k, jnp.sum(p), acc)
        x = acc + bu_v[pl.ds(o, 16)] + bm_v[pl.ds(o, 16)]
        y = 1.0 / (1.0 + jnp.exp(-x))
        out_v[pl.ds(o, 16)] = y * (1.0 + 2.0 * MARGIN) - MARGIN
        return carry

    lax.fori_loop(0, NWAVES, wave_body, 0)
    pltpu.sync_copy(out_v, out_hbm.at[pl.ds(base, BPW)])


@functools.partial(
    pl.kernel,
    out_type=jax.ShapeDtypeStruct((B,), jnp.float32),
    mesh=plsc.VectorSubcoreMesh(core_axis_name="c", subcore_axis_name="s"),
    compiler_params=pltpu.CompilerParams(
        needs_layout_passes=False,
        use_tc_tiling_on_sc=True,
    ),
    scratch_types=[
        pltpu.VMEM((BPW,), jnp.int32),
        pltpu.VMEM((BPW,), jnp.int32),
        pltpu.VMEM((2, 16, 8, D), jnp.float32),
        pltpu.VMEM((2, 16, 8, D), jnp.float32),
        pltpu.VMEM((BPW,), jnp.float32),
        pltpu.VMEM((BPW,), jnp.float32),
        pltpu.VMEM((BPW,), jnp.float32),
        pltpu.SemaphoreType.DMA,
        pltpu.SemaphoreType.DMA,
        pltpu.SemaphoreType.DMA,
    ],
)
def _cf_kernel(*args):
    _cf_body(*args)


def kernel(user_id, movie_id, emb_users, emb_movies, bias_user, bias_movie):
    return _cf_kernel(
        user_id.astype(jnp.int32),
        movie_id.astype(jnp.int32),
        emb_users.reshape(-1, 8, D),
        emb_movies.reshape(-1, 8, D),
        bias_user.reshape(-1),
        bias_movie.reshape(-1),
    )
